# baseline (speedup 1.0000x reference)
import functools

import jax
import jax.numpy as jnp
from jax import lax
from jax.experimental import pallas as pl
from jax.experimental.pallas import tpu as pltpu
from jax.experimental.pallas import tpu_sc as plsc

NUM_FREQ = 4
IN_DIM = 4
OUT_DIM = 4
NUM_BINS = 34
ROW = OUT_DIM * IN_DIM * NUM_FREQ
E = 800000

NC = 2
NS = 16
NW = NC * NS
L = 16

CH = 128
NCHUNK = E // CH
ITERS = -(-NCHUNK // NW)

_MAGIC = 8388608.0


def _bins_from_dists(d):
    x = jnp.clip((d - 2.4) / 0.4, 0.0, 33.0)
    r = (x + _MAGIC) - _MAGIC
    return r.astype(jnp.int32)


_mesh = plsc.VectorSubcoreMesh(core_axis_name="c", subcore_axis_name="s")


@functools.partial(
    pl.kernel,
    mesh=_mesh,
    out_type=jax.ShapeDtypeStruct((E, ROW), jnp.float32),
    scratch_types=[
        pltpu.VMEM((CH,), jnp.float32),
        pltpu.VMEM((CH,), jnp.int32),
        pltpu.VMEM((CH, ROW), jnp.float32),
        pltpu.SemaphoreType.DMA,
    ],
    compiler_params=pltpu.CompilerParams(use_tc_tiling_on_sc=False),
)
def _radial_sc(dists_hbm, table_hbm, out_hbm, d_v, idx_v, rows_v, sem):
    wid = lax.axis_index("s") * NC + lax.axis_index("c")

    def body(i, carry):
        c = wid + i * NW

        @pl.when(c < NCHUNK)
        def _():
            base = c * CH
            pltpu.sync_copy(dists_hbm.at[pl.ds(base, CH)], d_v)
            for k in range(CH // L):
                idx_v[pl.ds(k * L, L)] = _bins_from_dists(d_v[pl.ds(k * L, L)])
            pltpu.async_copy(table_hbm.at[idx_v], rows_v, sem).wait()
            pltpu.sync_copy(rows_v, out_hbm.at[pl.ds(base, CH)])

        return carry

    lax.fori_loop(0, ITERS, body, 0)


def kernel(dists, bin_embedding):
    flat = _radial_sc(dists.reshape(E), bin_embedding)
    return flat.reshape(E, OUT_DIM, 1, IN_DIM, 1, NUM_FREQ)
